# baseline (speedup 1.0000x reference)
import functools

import jax
import jax.numpy as jnp
from jax import lax
from jax.experimental import pallas as pl
from jax.experimental.pallas import tpu as pltpu
from jax.experimental.pallas import tpu_sc as plsc

L = 16
IDX_CHUNK = 128


def _lane_shuffle(v, perm):
    dnums = lax.GatherDimensionNumbers(
        offset_dims=(), collapsed_slice_dims=(0,), start_index_map=(0,))
    return lax.gather(v, perm.reshape(L, 1), dnums, slice_sizes=(1,),
                      mode=lax.GatherScatterMode.PROMISE_IN_BOUNDS)


def _neg_sqrt(s):
    i = lax.bitcast_convert_type(s, jnp.int32)
    i = jnp.int32(0x5F3759DF) - lax.shift_right_logical(i, 1)
    y = lax.bitcast_convert_type(i, jnp.float32)
    half_s = s * jnp.float32(0.5)
    for _ in range(3):
        y = y * (jnp.float32(1.5) - half_s * y * y)
    return -(s * y)


def _make_kernel(B, D, NC, NS):
    NW = NC * NS
    b_w = B // NW
    n_pass = b_w // IDX_CHUNK
    g_per_pass = IDX_CHUNK // L
    d_chunks = D // L
    D2 = 2 * D

    mesh = plsc.VectorSubcoreMesh(core_axis_name="c", subcore_axis_name="s")

    @functools.partial(
        pl.kernel,
        mesh=mesh,
        compiler_params=pltpu.CompilerParams(use_tc_tiling_on_sc=False),
        out_type=jax.ShapeDtypeStruct((B,), jnp.float32),
        scratch_types=[
            pltpu.VMEM((b_w,), jnp.int32),
            pltpu.VMEM((b_w,), jnp.int32),
            pltpu.VMEM((b_w,), jnp.int32),
            pltpu.VMEM((b_w,), jnp.int32),
            pltpu.VMEM((b_w,), jnp.int32),
            pltpu.VMEM((b_w,), jnp.int32),
            pltpu.VMEM((2 * IDX_CHUNK, D2), jnp.float32),
            pltpu.VMEM((2 * IDX_CHUNK, D2), jnp.float32),
            pltpu.VMEM((2 * IDX_CHUNK, D2), jnp.float32),
            pltpu.VMEM((b_w,), jnp.float32),
            pltpu.SemaphoreType.DMA,
        ],
    )
    def kge_kernel(head_hbm, rel_hbm, tail_hbm, node2_hbm, rel2_hbm,
                   out_hbm, ho, ro, to, h2i, r2i, t2i,
                   hbuf, rbuf, tbuf, out_v, sem):
        wid = lax.axis_index("s") * NC + lax.axis_index("c")
        base = wid * b_w

        pltpu.sync_copy(head_hbm.at[pl.ds(base, b_w)], ho)
        pltpu.sync_copy(rel_hbm.at[pl.ds(base, b_w)], ro)
        pltpu.sync_copy(tail_hbm.at[pl.ds(base, b_w)], to)

        for s in range(b_w // L):
            cs = pl.ds(s * L, L)
            h2i[cs] = lax.shift_right_logical(ho[cs], 1)
            r2i[cs] = lax.shift_right_logical(ro[cs], 1)
            t2i[cs] = lax.shift_right_logical(to[cs], 1)

        def issue(p):
            ps = pl.ds(p * IDX_CHUNK, IDX_CHUNK)
            bs = pl.ds((p % 2) * IDX_CHUNK, IDX_CHUNK)
            return [
                pltpu.async_copy(node2_hbm.at[h2i.at[ps]], hbuf.at[bs], sem),
                pltpu.async_copy(rel2_hbm.at[r2i.at[ps]], rbuf.at[bs], sem),
                pltpu.async_copy(node2_hbm.at[t2i.at[ps]], tbuf.at[bs], sem),
            ]

        lane_iota = lax.iota(jnp.int32, L)
        one = jnp.int32(1)

        def make_group_body(p):
            boff = (p % 2) * IDX_CHUNK

            def group_body(g, carry):
                rb = p * IDX_CHUNK + g * L
                lrb = boff + g * L
                hp = ho[pl.ds(rb, L)] & one
                rp = ro[pl.ds(rb, L)] & one
                tp = to[pl.ds(rb, L)] & one
                tot = None
                for r in range(L):
                    lrow = lrb + r
                    rsplat = jnp.full((L,), r, jnp.int32)
                    hm = _lane_shuffle(hp, rsplat).astype(jnp.float32)
                    rm = _lane_shuffle(rp, rsplat).astype(jnp.float32)
                    tm = _lane_shuffle(tp, rsplat).astype(jnp.float32)
                    acc = None
                    for c in range(d_chunks):
                        lo = pl.ds(c * L, L)
                        hi = pl.ds(D + c * L, L)
                        hl = hbuf[lrow, lo]
                        rl = rbuf[lrow, lo]
                        tl = tbuf[lrow, lo]
                        hv = hl + hm * (hbuf[lrow, hi] - hl)
                        rv = rl + rm * (rbuf[lrow, hi] - rl)
                        tv = tl + tm * (tbuf[lrow, hi] - tl)
                        d = hv + rv - tv
                        acc = d * d if acc is None else acc + d * d
                    for step in (8, 4, 2, 1):
                        acc = acc + _lane_shuffle(acc, lane_iota ^ step)
                    sel = lane_iota == jnp.int32(r)
                    tot = acc if r == 0 else jnp.where(sel, acc, tot)
                out_v[pl.ds(rb, L)] = _neg_sqrt(tot + jnp.float32(1e-12))
                return carry

            return group_body

        copies = issue(0)
        for p in range(n_pass):
            for c in copies:
                c.wait()
            if p + 1 < n_pass:
                copies = issue(p + 1)
            lax.fori_loop(0, g_per_pass, make_group_body(p), 0)

        pltpu.sync_copy(out_v, out_hbm.at[pl.ds(base, b_w)])

    return kge_kernel


def kernel(head_index, rel_type, tail_index, node_emb, rel_emb):
    B = head_index.shape[0]
    D = node_emb.shape[1]
    info = plsc.get_sparse_core_info()
    k = _make_kernel(B, D, info.num_cores, info.num_subcores)
    node2 = node_emb.reshape(-1, 2 * D)
    rel2 = rel_emb.reshape(-1, 2 * D)
    return k(head_index.astype(jnp.int32), rel_type.astype(jnp.int32),
             tail_index.astype(jnp.int32), node2, rel2)


# speedup vs baseline: 1.9962x; 1.9962x over previous
import functools

import jax
import jax.numpy as jnp
from jax import lax
from jax.experimental import pallas as pl
from jax.experimental.pallas import tpu as pltpu
from jax.experimental.pallas import tpu_sc as plsc

L = 16
SUB = 8


def _lane_shuffle(v, perm):
    dnums = lax.GatherDimensionNumbers(
        offset_dims=(), collapsed_slice_dims=(0,), start_index_map=(0,))
    return lax.gather(v, perm.reshape(L, 1), dnums, slice_sizes=(1,),
                      mode=lax.GatherScatterMode.PROMISE_IN_BOUNDS)


def _neg_sqrt(s):
    i = lax.bitcast_convert_type(s, jnp.int32)
    i = jnp.int32(0x5F3759DF) - lax.shift_right_logical(i, 1)
    y = lax.bitcast_convert_type(i, jnp.float32)
    half_s = s * jnp.float32(0.5)
    for _ in range(3):
        y = y * (jnp.float32(1.5) - half_s * y * y)
    return -(s * y)


def _make_kernel(B, D, NC, NS):
    NW = NC * NS
    b_w = B // NW
    n_groups = b_w // L
    d_chunks = D // L

    mesh = plsc.VectorSubcoreMesh(core_axis_name="c", subcore_axis_name="s")

    tile_t = pltpu.VMEM((L, SUB, D), jnp.float32)

    @functools.partial(
        pl.kernel,
        mesh=mesh,
        compiler_params=pltpu.CompilerParams(use_tc_tiling_on_sc=True),
        out_type=jax.ShapeDtypeStruct((B,), jnp.float32),
        scratch_types=[
            pltpu.VMEM((b_w,), jnp.int32),
            pltpu.VMEM((b_w,), jnp.int32),
            pltpu.VMEM((b_w,), jnp.int32),
            tile_t, tile_t,
            tile_t, tile_t,
            tile_t, tile_t,
            pltpu.VMEM((b_w,), jnp.float32),
            pltpu.SemaphoreType.DMA,
            pltpu.SemaphoreType.DMA,
        ],
    )
    def kge_kernel(head_hbm, rel_hbm, tail_hbm, node3_hbm, rel3_hbm,
                   out_hbm, ho, ro, to, hA, hB, rA, rB, tA, tB,
                   out_v, semA, semB):
        wid = lax.axis_index("s") * NC + lax.axis_index("c")
        base = wid * b_w

        pltpu.sync_copy(head_hbm.at[pl.ds(base, b_w)], ho)
        pltpu.sync_copy(rel_hbm.at[pl.ds(base, b_w)], ro)
        pltpu.sync_copy(tail_hbm.at[pl.ds(base, b_w)], to)

        seven = jnp.int32(7)
        three = jnp.int32(3)
        lane_iota = lax.iota(jnp.int32, L)

        def issue(g, bufs, sem):
            hbuf, rbuf, tbuf = bufs
            hv = ho[pl.ds(g * L, L)]
            rv = ro[pl.ds(g * L, L)]
            tv = to[pl.ds(g * L, L)]
            for r in range(L):
                pltpu.async_copy(
                    node3_hbm.at[lax.shift_right_logical(hv[r], three)],
                    hbuf.at[r], sem)
                pltpu.async_copy(
                    rel3_hbm.at[lax.shift_right_logical(rv[r], three)],
                    rbuf.at[r], sem)
                pltpu.async_copy(
                    node3_hbm.at[lax.shift_right_logical(tv[r], three)],
                    tbuf.at[r], sem)

        def drain(bufs, sem):
            for buf in bufs:
                pltpu.make_async_copy(
                    node3_hbm.at[pl.ds(0, L)], buf, sem).wait()

        def compute(g, bufs):
            hbuf, rbuf, tbuf = bufs
            hv = ho[pl.ds(g * L, L)]
            rv = ro[pl.ds(g * L, L)]
            tv = to[pl.ds(g * L, L)]
            tot = None
            for r in range(L):
                hsub = hv[r] & seven
                rsub = rv[r] & seven
                tsub = tv[r] & seven
                acc = None
                for c in range(d_chunks):
                    cs = pl.ds(c * L, L)
                    d = (hbuf[r, hsub, cs] + rbuf[r, rsub, cs]
                         - tbuf[r, tsub, cs])
                    acc = d * d if acc is None else acc + d * d
                for step in (8, 4, 2, 1):
                    acc = acc + _lane_shuffle(acc, lane_iota ^ step)
                sel = lane_iota == jnp.int32(r)
                tot = acc if r == 0 else jnp.where(sel, acc, tot)
            out_v[pl.ds(g * L, L)] = _neg_sqrt(tot + jnp.float32(1e-12))

        A = (hA, rA, tA)
        Bb = (hB, rB, tB)
        issue(0, A, semA)

        def body(k, carry):
            g0 = 2 * k
            g1 = g0 + 1
            issue(g1, Bb, semB)
            drain(A, semA)
            compute(g0, A)

            @pl.when(g1 + 1 < n_groups)
            def _():
                issue(g1 + 1, A, semA)

            drain(Bb, semB)
            compute(g1, Bb)
            return carry

        lax.fori_loop(0, n_groups // 2, body, 0)
        pltpu.sync_copy(out_v, out_hbm.at[pl.ds(base, b_w)])

    return kge_kernel


def kernel(head_index, rel_type, tail_index, node_emb, rel_emb):
    B = head_index.shape[0]
    D = node_emb.shape[1]
    info = plsc.get_sparse_core_info()
    k = _make_kernel(B, D, info.num_cores, info.num_subcores)
    node3 = node_emb.reshape(-1, SUB, D)
    rel3 = rel_emb.reshape(-1, SUB, D)
    return k(head_index.astype(jnp.int32), rel_type.astype(jnp.int32),
             tail_index.astype(jnp.int32), node3, rel3)


# speedup vs baseline: 2.1553x; 1.0797x over previous
import functools

import jax
import jax.numpy as jnp
from jax import lax
from jax.experimental import pallas as pl
from jax.experimental.pallas import tpu as pltpu
from jax.experimental.pallas import tpu_sc as plsc

L = 16
SUB = 8


def _lane_shuffle(v, perm):
    dnums = lax.GatherDimensionNumbers(
        offset_dims=(), collapsed_slice_dims=(0,), start_index_map=(0,))
    return lax.gather(v, perm.reshape(L, 1), dnums, slice_sizes=(1,),
                      mode=lax.GatherScatterMode.PROMISE_IN_BOUNDS)


def _neg_sqrt(s):
    i = lax.bitcast_convert_type(s, jnp.int32)
    i = jnp.int32(0x5F3759DF) - lax.shift_right_logical(i, 1)
    y = lax.bitcast_convert_type(i, jnp.float32)
    half_s = s * jnp.float32(0.5)
    for _ in range(3):
        y = y * (jnp.float32(1.5) - half_s * y * y)
    return -(s * y)


def _make_kernel(B, D, NC, NS):
    NW = NC * NS
    b_w = B // NW
    n_groups = b_w // L
    d_chunks = D // L

    mesh = plsc.VectorSubcoreMesh(core_axis_name="c", subcore_axis_name="s")

    tile_t = pltpu.VMEM((L, SUB, D), jnp.float32)

    @functools.partial(
        pl.kernel,
        mesh=mesh,
        compiler_params=pltpu.CompilerParams(use_tc_tiling_on_sc=True),
        out_type=jax.ShapeDtypeStruct((B,), jnp.float32),
        scratch_types=[
            pltpu.VMEM((b_w,), jnp.int32),
            pltpu.VMEM((b_w,), jnp.int32),
            pltpu.VMEM((b_w,), jnp.int32),
            pltpu.VMEM((b_w,), jnp.int32),
            tile_t, tile_t,
            pltpu.VMEM((L, 2 * D), jnp.float32),
            pltpu.VMEM((L, 2 * D), jnp.float32),
            tile_t, tile_t,
            pltpu.VMEM((b_w,), jnp.float32),
            pltpu.SemaphoreType.DMA,
            pltpu.SemaphoreType.DMA,
        ],
    )
    def kge_kernel(head_hbm, rel_hbm, tail_hbm, node3_hbm, rel2_hbm,
                   out_hbm, ho, ro, to, r2i, hA, hB, rA, rB, tA, tB,
                   out_v, semA, semB):
        wid = lax.axis_index("s") * NC + lax.axis_index("c")
        base = wid * b_w

        pltpu.sync_copy(head_hbm.at[pl.ds(base, b_w)], ho)
        pltpu.sync_copy(rel_hbm.at[pl.ds(base, b_w)], ro)
        pltpu.sync_copy(tail_hbm.at[pl.ds(base, b_w)], to)

        for s in range(b_w // L):
            cs = pl.ds(s * L, L)
            r2i[cs] = lax.shift_right_logical(ro[cs], 1)

        seven = jnp.int32(7)
        three = jnp.int32(3)
        lane_iota = lax.iota(jnp.int32, L)

        def issue(g, bufs, sem):
            hbuf, rbuf, tbuf = bufs
            hv = ho[pl.ds(g * L, L)]
            tv = to[pl.ds(g * L, L)]
            pltpu.async_copy(
                rel2_hbm.at[r2i.at[pl.ds(g * L, L)]], rbuf, sem)
            for r in range(L):
                pltpu.async_copy(
                    node3_hbm.at[lax.shift_right_logical(hv[r], three)],
                    hbuf.at[r], sem)
                pltpu.async_copy(
                    node3_hbm.at[lax.shift_right_logical(tv[r], three)],
                    tbuf.at[r], sem)

        def drain(bufs, sem):
            hbuf, rbuf, tbuf = bufs
            pltpu.make_async_copy(node3_hbm.at[pl.ds(0, L)], hbuf, sem).wait()
            pltpu.make_async_copy(node3_hbm.at[pl.ds(0, L)], tbuf, sem).wait()
            pltpu.make_async_copy(rel2_hbm.at[pl.ds(0, L)], rbuf, sem).wait()

        def compute(g, bufs):
            hbuf, rbuf, tbuf = bufs
            hv = ho[pl.ds(g * L, L)]
            tv = to[pl.ds(g * L, L)]
            rp = (ro[pl.ds(g * L, L)] & jnp.int32(1)).astype(jnp.float32)
            tot = None
            for r in range(L):
                hsub = hv[r] & seven
                tsub = tv[r] & seven
                rm = _lane_shuffle(rp, jnp.full((L,), r, jnp.int32))
                acc = None
                for c in range(d_chunks):
                    cs = pl.ds(c * L, L)
                    hs = pl.ds(D + c * L, L)
                    rl = rbuf[r, cs]
                    rv = rl + rm * (rbuf[r, hs] - rl)
                    d = hbuf[r, hsub, cs] + rv - tbuf[r, tsub, cs]
                    acc = d * d if acc is None else acc + d * d
                for step in (8, 4, 2, 1):
                    acc = acc + _lane_shuffle(acc, lane_iota ^ step)
                sel = lane_iota == jnp.int32(r)
                tot = acc if r == 0 else jnp.where(sel, acc, tot)
            out_v[pl.ds(g * L, L)] = _neg_sqrt(tot + jnp.float32(1e-12))

        A = (hA, rA, tA)
        Bb = (hB, rB, tB)
        issue(0, A, semA)

        def body(k, carry):
            g0 = 2 * k
            g1 = g0 + 1
            issue(g1, Bb, semB)
            drain(A, semA)
            compute(g0, A)

            @pl.when(g1 + 1 < n_groups)
            def _():
                issue(g1 + 1, A, semA)

            drain(Bb, semB)
            compute(g1, Bb)
            return carry

        lax.fori_loop(0, n_groups // 2, body, 0)
        pltpu.sync_copy(out_v, out_hbm.at[pl.ds(base, b_w)])

    return kge_kernel


def kernel(head_index, rel_type, tail_index, node_emb, rel_emb):
    B = head_index.shape[0]
    D = node_emb.shape[1]
    info = plsc.get_sparse_core_info()
    k = _make_kernel(B, D, info.num_cores, info.num_subcores)
    node3 = node_emb.reshape(-1, SUB, D)
    rel2 = rel_emb.reshape(-1, 2 * D)
    return k(head_index.astype(jnp.int32), rel_type.astype(jnp.int32),
             tail_index.astype(jnp.int32), node3, rel2)
